# Initial kernel scaffold; baseline (speedup 1.0000x reference)
import jax
import jax.numpy as jnp
from jax.experimental import pallas as pl


def kernel(x, W1, g1, b1, W2, g2, b2, W3, g3, b3, W4, g4, b4, gcat, bcat, Wg, gg, bg, Wc1, gc1, bc1, Wc2, gc2, bc2, Wc3, bc3):
    raise NotImplementedError("write your pallas kernel here")


# speedup vs baseline: 1.7927x; 1.7927x over previous
import functools

import jax
import jax.numpy as jnp
from jax import lax
from jax.experimental import pallas as pl
from jax.experimental.pallas import tpu as pltpu
from jax.experimental.pallas import tpu_sc as plsc

K = 20
EPS = 1e-5
N_PTS = 2048
NB = 4
BN = NB * N_PTS
NW = 32
PTS_W = BN // NW
RCH = 80
TN = 64


def _make_sc_gather_rows(Cp):
    mesh = plsc.VectorSubcoreMesh(core_axis_name="c", subcore_axis_name="s")
    ew = PTS_W * K

    @functools.partial(
        pl.kernel, mesh=mesh,
        compiler_params=pltpu.CompilerParams(
            needs_layout_passes=False, use_tc_tiling_on_sc=False),
        out_type=jax.ShapeDtypeStruct((BN * K, Cp), jnp.float32),
        scratch_types=[
            pltpu.VMEM((ew,), jnp.int32),
            pltpu.VMEM((RCH, Cp), jnp.float32),
            pltpu.VMEM((RCH, Cp), jnp.float32),
            pltpu.SemaphoreType.DMA,
            pltpu.SemaphoreType.DMA,
        ],
    )
    def sc_gather(idx_hbm, tab_hbm, g_hbm, idx_v, rows0_v, rows1_v, sem0, sem1):
        wid = lax.axis_index("s") * 2 + lax.axis_index("c")
        ebase = wid * ew
        pltpu.sync_copy(idx_hbm.at[pl.ds(ebase, ew)], idx_v)
        nch = ew // RCH
        rows = (rows0_v, rows1_v)
        sems = (sem0, sem1)

        def gath(c, buf):
            return pltpu.async_copy(
                tab_hbm.at[idx_v.at[pl.ds(c * RCH, RCH)]], rows[buf], sems[buf])

        def drain(buf):
            pltpu.make_async_copy(
                tab_hbm.at[idx_v.at[pl.ds(0, RCH)]], rows[buf], sems[buf]).wait()

        gath(0, 0)

        def _chunk(c, _):
            def body2(par):
                cc = c * 2 + par
                drain(par)
                nxt = cc + 1

                @pl.when(nxt < nch)
                def _():
                    gath(nxt, 1 - par)
                pltpu.sync_copy(rows[par], g_hbm.at[pl.ds(ebase + cc * RCH, RCH)])
            body2(0)
            body2(1)
            return _
        lax.fori_loop(0, nch // 2, _chunk, 0)

    return sc_gather


_SC_CACHE = {}


def _sc_gather_rows(idx_glob, table, Cp):
    if Cp not in _SC_CACHE:
        _SC_CACHE[Cp] = _make_sc_gather_rows(Cp)
    return _SC_CACHE[Cp](idx_glob, table)


def _make_edge_conv(Cp, O):
    grid = BN // TN

    def body(xi_ref, g_ref, w_ref, m_ref, y_ref):
        xi = xi_ref[...]
        gj = g_ref[...]
        diff = gj.reshape(TN, K, Cp) - xi[:, None, :]
        fa = diff.astype(jnp.bfloat16).reshape(TN * K, Cp)
        xb = jnp.broadcast_to(
            xi.astype(jnp.bfloat16)[:, None, :], (TN, K, Cp)).reshape(TN * K, Cp)
        f = jnp.concatenate([fa, xb], axis=1)
        w = w_ref[...]
        y = jax.lax.dot_general(
            f, w, (((1,), (0,)), ((), ())),
            preferred_element_type=jnp.float32)
        m_ref[...] = jnp.max(y.reshape(TN, K, O), axis=1)
        y_ref[...] = y

    return pl.pallas_call(
        body,
        grid=(grid,),
        in_specs=[
            pl.BlockSpec((TN, Cp), lambda i: (i, 0)),
            pl.BlockSpec((TN * K, Cp), lambda i: (i, 0)),
            pl.BlockSpec((2 * Cp, O), lambda i: (0, 0)),
        ],
        out_specs=[
            pl.BlockSpec((TN, O), lambda i: (i, 0)),
            pl.BlockSpec((TN * K, O), lambda i: (i, 0)),
        ],
        out_shape=[
            jax.ShapeDtypeStruct((BN, O), jnp.float32),
            jax.ShapeDtypeStruct((BN * K, O), jnp.float32),
        ],
    )


def _make_finalize(O):
    grid = BN // 512

    def body(m_ref, mu_ref, var_ref, g_ref, b_ref, o_ref, t_ref):
        mean = mu_ref[...]
        scale = g_ref[...] / jnp.sqrt(var_ref[...] + EPS)
        shift = b_ref[...] - mean * scale
        v = m_ref[...] * scale + shift
        v = jnp.where(v > 0, v, 0.2 * v)
        o_ref[...] = v
        t_ref[...] = jnp.transpose(v)[None]

    return pl.pallas_call(
        body,
        grid=(grid,),
        in_specs=[
            pl.BlockSpec((512, O), lambda i: (i, 0)),
            pl.BlockSpec((1, O), lambda i: (0, 0)),
            pl.BlockSpec((1, O), lambda i: (0, 0)),
            pl.BlockSpec((1, O), lambda i: (0, 0)),
            pl.BlockSpec((1, O), lambda i: (0, 0)),
        ],
        out_specs=[
            pl.BlockSpec((512, O), lambda i: (i, 0)),
            pl.BlockSpec((1, O, 512), lambda i: (i // (N_PTS // 512), 0,
                                                 i % (N_PTS // 512))),
        ],
        out_shape=[
            jax.ShapeDtypeStruct((BN, O), jnp.float32),
            jax.ShapeDtypeStruct((NB, O, N_PTS), jnp.float32),
        ],
    )


_EC_CACHE = {}


def _edge_conv(xt_pad, G, w_t, g, b, Cp, O):
    if (Cp, O) not in _EC_CACHE:
        _EC_CACHE[(Cp, O)] = (_make_edge_conv(Cp, O), _make_finalize(O))
    conv, fin = _EC_CACHE[(Cp, O)]
    M, Y = conv(xt_pad, G, w_t)
    y4 = jnp.transpose(Y.reshape(NB, N_PTS, K, O), (0, 3, 1, 2))
    mean = jnp.mean(y4, axis=(0, 2, 3))
    var = jnp.var(y4, axis=(0, 2, 3))
    return fin(M, mean.reshape(1, O), var.reshape(1, O),
               g.reshape(1, O), b.reshape(1, O))


def _knn_idx(x):
    inner = -2.0 * jnp.einsum('bcn,bcm->bnm', x, x)
    xx = jnp.sum(x * x, axis=1)
    pairwise = -xx[:, :, None] - inner - xx[:, None, :]
    _, idx = jax.lax.top_k(pairwise, K)
    return idx


def _edge_block(xt_pad, x_cm, W, g, b, C, Cp):
    O = W.shape[0]
    idx = _knn_idx(x_cm)
    boffs = (jnp.arange(NB, dtype=jnp.int32) * N_PTS)[:, None, None]
    idx_glob = (idx + boffs).reshape(-1)
    G = _sc_gather_rows(idx_glob, xt_pad.reshape(BN, Cp), Cp)
    Wd, Wc = W[:, :C], W[:, C:]
    pad = Cp - C
    if pad:
        Wd = jnp.pad(Wd, ((0, 0), (0, pad)))
        Wc = jnp.pad(Wc, ((0, 0), (0, pad)))
    w_t = jnp.concatenate([Wd.T, Wc.T], axis=0).astype(jnp.bfloat16)
    return _edge_conv(xt_pad.reshape(BN, Cp), G, w_t, g, b, Cp, O)


def _lrelu(x):
    return jnp.where(x > 0, x, 0.2 * x)


def _edge_block_exact(xt_pad, x_cm, W, g, b, C, Cp):
    O = W.shape[0]
    idx = _knn_idx(x_cm)
    boffs = (jnp.arange(NB, dtype=jnp.int32) * N_PTS)[:, None, None]
    idx_glob = (idx + boffs).reshape(-1)
    G = _sc_gather_rows(idx_glob, xt_pad.reshape(BN, Cp), Cp)
    feature = G.reshape(NB, N_PTS, K, Cp)[:, :, :, :C]
    center = jnp.broadcast_to(
        xt_pad[:, :, None, :C], (NB, N_PTS, K, C))
    f = jnp.concatenate([feature - center, center], axis=-1)
    f = jnp.transpose(f, (0, 3, 1, 2))
    y = jnp.einsum('oc,bcnk->bonk', W, f)
    mean = jnp.mean(y, axis=(0, 2, 3), keepdims=True)
    var = jnp.var(y, axis=(0, 2, 3), keepdims=True)
    xhat = (y - mean) / jnp.sqrt(var + EPS)
    yn = xhat * g.reshape(1, O, 1, 1) + b.reshape(1, O, 1, 1)
    out_cm = jnp.max(_lrelu(yn), axis=-1)
    return jnp.transpose(out_cm, (0, 2, 1)), out_cm


def _bn_pw(x, g, b):
    mean = jnp.mean(x, axis=(0, 1), keepdims=True)
    var = jnp.var(x, axis=(0, 1), keepdims=True)
    return (x - mean) / jnp.sqrt(var + EPS) * g[None, None, :] + b[None, None, :]


def kernel(x, W1, g1, b1, W2, g2, b2, W3, g3, b3, W4, g4, b4, gcat, bcat,
           Wg, gg, bg, Wc1, gc1, bc1, Wc2, gc2, bc2, Wc3, bc3):
    B, C, N = x.shape
    xt = jnp.transpose(x, (0, 2, 1))
    xt_pad = jnp.pad(xt, ((0, 0), (0, 0), (0, 16 - C)))
    x1, x1_cm = _edge_block_exact(xt_pad, x, W1, g1, b1, C, 16)
    x2, x2_cm = _edge_block_exact(x1, x1_cm, W2, g2, b2, 64, 64)
    x3, x3_cm = _edge_block_exact(x2, x2_cm, W3, g3, b3, 64, 64)
    x4, _ = _edge_block(x3, x3_cm, W4, g4, b4, 128, 128)
    x4 = x4.reshape(B, N, 256)
    x_local = jnp.concatenate([x1, x2, x3, x4], axis=2)
    x_local = _bn_pw(x_local, gcat, bcat)
    xg = _lrelu(_bn_pw(jnp.einsum('bnc,oc->bno', x_local, Wg), gg, bg))
    x_global = jnp.max(xg, axis=1)
    x_cat = jnp.concatenate(
        [x_local, jnp.broadcast_to(x_global[:, None, :], (B, N, 1024))], axis=2)
    h = _lrelu(_bn_pw(jnp.einsum('bnc,oc->bno', x_cat, Wc1), gc1, bc1))
    h = _lrelu(_bn_pw(jnp.einsum('bnc,oc->bno', h, Wc2), gc2, bc2))
    logits = jnp.einsum('bnc,oc->bno', h, Wc3) + bc3[None, None, :]
    return (jnp.transpose(logits, (0, 2, 1)),
            jnp.transpose(x_local, (0, 2, 1)),
            x_global[:, :, None])


# speedup vs baseline: 1.8335x; 1.0228x over previous
import functools

import jax
import jax.numpy as jnp
from jax import lax
from jax.experimental import pallas as pl
from jax.experimental.pallas import tpu as pltpu
from jax.experimental.pallas import tpu_sc as plsc

K = 20
EPS = 1e-5
N_PTS = 2048
NB = 4
BN = NB * N_PTS
NW = 32
PTS_W = BN // NW
RCH = 80
TN = 64


def _make_sc_gather_rows(Cp):
    mesh = plsc.VectorSubcoreMesh(core_axis_name="c", subcore_axis_name="s")
    ew = PTS_W * K

    @functools.partial(
        pl.kernel, mesh=mesh,
        compiler_params=pltpu.CompilerParams(
            needs_layout_passes=False, use_tc_tiling_on_sc=False),
        out_type=jax.ShapeDtypeStruct((BN * K, Cp), jnp.float32),
        scratch_types=[
            pltpu.VMEM((ew,), jnp.int32),
            pltpu.VMEM((RCH, Cp), jnp.float32),
            pltpu.VMEM((RCH, Cp), jnp.float32),
            pltpu.SemaphoreType.DMA,
            pltpu.SemaphoreType.DMA,
        ],
    )
    def sc_gather(idx_hbm, tab_hbm, g_hbm, idx_v, rows0_v, rows1_v, sem0, sem1):
        wid = lax.axis_index("s") * 2 + lax.axis_index("c")
        ebase = wid * ew
        pltpu.sync_copy(idx_hbm.at[pl.ds(ebase, ew)], idx_v)
        nch = ew // RCH
        rows = (rows0_v, rows1_v)
        sems = (sem0, sem1)

        def gath(c, buf):
            return pltpu.async_copy(
                tab_hbm.at[idx_v.at[pl.ds(c * RCH, RCH)]], rows[buf], sems[buf])

        def drain(buf):
            pltpu.make_async_copy(
                tab_hbm.at[idx_v.at[pl.ds(0, RCH)]], rows[buf], sems[buf]).wait()

        gath(0, 0)

        def _chunk(c, _):
            def body2(par):
                cc = c * 2 + par
                drain(par)
                nxt = cc + 1

                @pl.when(nxt < nch)
                def _():
                    gath(nxt, 1 - par)
                pltpu.sync_copy(rows[par], g_hbm.at[pl.ds(ebase + cc * RCH, RCH)])
            body2(0)
            body2(1)
            return _
        lax.fori_loop(0, nch // 2, _chunk, 0)

    return sc_gather


_SC_CACHE = {}


def _sc_gather_rows(idx_glob, table, Cp):
    if Cp not in _SC_CACHE:
        _SC_CACHE[Cp] = _make_sc_gather_rows(Cp)
    return _SC_CACHE[Cp](idx_glob, table)


def _make_edge_conv(Cp, O):
    grid = BN // TN

    def body(xi_ref, g_ref, w_ref, m_ref, s1_ref, s2_ref, c1_ref, c2_ref):
        step = pl.program_id(0)
        xi = xi_ref[...]
        gj = g_ref[...]
        diff = gj.reshape(TN, K, Cp) - xi[:, None, :]
        fa = diff.astype(jnp.bfloat16).reshape(TN * K, Cp)
        xb = jnp.broadcast_to(
            xi.astype(jnp.bfloat16)[:, None, :], (TN, K, Cp)).reshape(TN * K, Cp)
        f = jnp.concatenate([fa, xb], axis=1)
        w = w_ref[...]
        y = jax.lax.dot_general(
            f, w, (((1,), (0,)), ((), ())),
            preferred_element_type=jnp.float32)
        m_ref[...] = jnp.max(y.reshape(TN, K, O), axis=1)
        ps = jnp.sum(y, axis=0)[None]
        psq = jnp.sum(y * y, axis=0)[None]

        @pl.when(step == 0)
        def _():
            s1_ref[...] = ps
            s2_ref[...] = psq
            c1_ref[...] = jnp.zeros_like(ps)
            c2_ref[...] = jnp.zeros_like(ps)

        @pl.when(step != 0)
        def _():
            def acc(s_ref, c_ref, v):
                v = v + c_ref[...]
                s = s_ref[...]
                t = s + v
                c_ref[...] = (s - t) + v
                s_ref[...] = t
            acc(s1_ref, c1_ref, ps)
            acc(s2_ref, c2_ref, psq)

    return pl.pallas_call(
        body,
        grid=(grid,),
        in_specs=[
            pl.BlockSpec((TN, Cp), lambda i: (i, 0)),
            pl.BlockSpec((TN * K, Cp), lambda i: (i, 0)),
            pl.BlockSpec((2 * Cp, O), lambda i: (0, 0)),
        ],
        out_specs=[
            pl.BlockSpec((TN, O), lambda i: (i, 0)),
            pl.BlockSpec((1, O), lambda i: (0, 0)),
            pl.BlockSpec((1, O), lambda i: (0, 0)),
        ],
        out_shape=[
            jax.ShapeDtypeStruct((BN, O), jnp.float32),
            jax.ShapeDtypeStruct((1, O), jnp.float32),
            jax.ShapeDtypeStruct((1, O), jnp.float32),
        ],
        scratch_shapes=[
            pltpu.VMEM((1, O), jnp.float32),
            pltpu.VMEM((1, O), jnp.float32),
        ],
    )


def _make_finalize(O):
    grid = BN // 512

    def body(m_ref, mu_ref, var_ref, g_ref, b_ref, o_ref, t_ref):
        mean = mu_ref[...]
        scale = g_ref[...] / jnp.sqrt(var_ref[...] + EPS)
        shift = b_ref[...] - mean * scale
        v = m_ref[...] * scale + shift
        v = jnp.where(v > 0, v, 0.2 * v)
        o_ref[...] = v
        t_ref[...] = jnp.transpose(v)[None]

    return pl.pallas_call(
        body,
        grid=(grid,),
        in_specs=[
            pl.BlockSpec((512, O), lambda i: (i, 0)),
            pl.BlockSpec((1, O), lambda i: (0, 0)),
            pl.BlockSpec((1, O), lambda i: (0, 0)),
            pl.BlockSpec((1, O), lambda i: (0, 0)),
            pl.BlockSpec((1, O), lambda i: (0, 0)),
        ],
        out_specs=[
            pl.BlockSpec((512, O), lambda i: (i, 0)),
            pl.BlockSpec((1, O, 512), lambda i: (i // (N_PTS // 512), 0,
                                                 i % (N_PTS // 512))),
        ],
        out_shape=[
            jax.ShapeDtypeStruct((BN, O), jnp.float32),
            jax.ShapeDtypeStruct((NB, O, N_PTS), jnp.float32),
        ],
    )


_EC_CACHE = {}


def _edge_conv(xt_pad, G, w_t, g, b, Cp, O):
    if (Cp, O) not in _EC_CACHE:
        _EC_CACHE[(Cp, O)] = (_make_edge_conv(Cp, O), _make_finalize(O))
    conv, fin = _EC_CACHE[(Cp, O)]
    M, s1, s2 = conv(xt_pad, G, w_t)
    E = float(BN * K)
    mean = s1 / E
    var = s2 / E - mean * mean
    return fin(M, mean, var, g.reshape(1, O), b.reshape(1, O))


def _knn_idx(x):
    inner = -2.0 * jnp.einsum('bcn,bcm->bnm', x, x)
    xx = jnp.sum(x * x, axis=1)
    pairwise = -xx[:, :, None] - inner - xx[:, None, :]
    _, idx = jax.lax.top_k(pairwise, K)
    return idx


def _edge_block(xt_pad, x_cm, W, g, b, C, Cp):
    O = W.shape[0]
    idx = _knn_idx(x_cm)
    boffs = (jnp.arange(NB, dtype=jnp.int32) * N_PTS)[:, None, None]
    idx_glob = (idx + boffs).reshape(-1)
    G = _sc_gather_rows(idx_glob, xt_pad.reshape(BN, Cp), Cp)
    Wd, Wc = W[:, :C], W[:, C:]
    pad = Cp - C
    if pad:
        Wd = jnp.pad(Wd, ((0, 0), (0, pad)))
        Wc = jnp.pad(Wc, ((0, 0), (0, pad)))
    w_t = jnp.concatenate([Wd.T, Wc.T], axis=0).astype(jnp.bfloat16)
    return _edge_conv(xt_pad.reshape(BN, Cp), G, w_t, g, b, Cp, O)


def _lrelu(x):
    return jnp.where(x > 0, x, 0.2 * x)


def _edge_block_exact(xt_pad, x_cm, W, g, b, C, Cp):
    O = W.shape[0]
    idx = _knn_idx(x_cm)
    boffs = (jnp.arange(NB, dtype=jnp.int32) * N_PTS)[:, None, None]
    idx_glob = (idx + boffs).reshape(-1)
    G = _sc_gather_rows(idx_glob, xt_pad.reshape(BN, Cp), Cp)
    feature = G.reshape(NB, N_PTS, K, Cp)[:, :, :, :C]
    center = jnp.broadcast_to(
        xt_pad[:, :, None, :C], (NB, N_PTS, K, C))
    f = jnp.concatenate([feature - center, center], axis=-1)
    f = jnp.transpose(f, (0, 3, 1, 2))
    y = jnp.einsum('oc,bcnk->bonk', W, f)
    mean = jnp.mean(y, axis=(0, 2, 3), keepdims=True)
    var = jnp.var(y, axis=(0, 2, 3), keepdims=True)
    xhat = (y - mean) / jnp.sqrt(var + EPS)
    yn = xhat * g.reshape(1, O, 1, 1) + b.reshape(1, O, 1, 1)
    out_cm = jnp.max(_lrelu(yn), axis=-1)
    return jnp.transpose(out_cm, (0, 2, 1)), out_cm


def _bn_pw(x, g, b):
    mean = jnp.mean(x, axis=(0, 1), keepdims=True)
    var = jnp.var(x, axis=(0, 1), keepdims=True)
    return (x - mean) / jnp.sqrt(var + EPS) * g[None, None, :] + b[None, None, :]


def kernel(x, W1, g1, b1, W2, g2, b2, W3, g3, b3, W4, g4, b4, gcat, bcat,
           Wg, gg, bg, Wc1, gc1, bc1, Wc2, gc2, bc2, Wc3, bc3):
    B, C, N = x.shape
    xt = jnp.transpose(x, (0, 2, 1))
    xt_pad = jnp.pad(xt, ((0, 0), (0, 0), (0, 16 - C)))
    x1, x1_cm = _edge_block_exact(xt_pad, x, W1, g1, b1, C, 16)
    x2, x2_cm = _edge_block_exact(x1, x1_cm, W2, g2, b2, 64, 64)
    x3, x3_cm = _edge_block_exact(x2, x2_cm, W3, g3, b3, 64, 64)
    x4, _ = _edge_block(x3, x3_cm, W4, g4, b4, 128, 128)
    x4 = x4.reshape(B, N, 256)
    x_local = jnp.concatenate([x1, x2, x3, x4], axis=2)
    x_local = _bn_pw(x_local, gcat, bcat)
    xg = _lrelu(_bn_pw(jnp.einsum('bnc,oc->bno', x_local, Wg), gg, bg))
    x_global = jnp.max(xg, axis=1)
    x_cat = jnp.concatenate(
        [x_local, jnp.broadcast_to(x_global[:, None, :], (B, N, 1024))], axis=2)
    h = _lrelu(_bn_pw(jnp.einsum('bnc,oc->bno', x_cat, Wc1), gc1, bc1))
    h = _lrelu(_bn_pw(jnp.einsum('bnc,oc->bno', h, Wc2), gc2, bc2))
    logits = jnp.einsum('bnc,oc->bno', h, Wc3) + bc3[None, None, :]
    return (jnp.transpose(logits, (0, 2, 1)),
            jnp.transpose(x_local, (0, 2, 1)),
            x_global[:, :, None])


# speedup vs baseline: 4.7906x; 2.6129x over previous
import functools

import jax
import jax.numpy as jnp
from jax import lax
from jax.experimental import pallas as pl
from jax.experimental.pallas import tpu as pltpu
from jax.experimental.pallas import tpu_sc as plsc

K = 20
EPS = 1e-5
N_PTS = 2048
NB = 4
BN = NB * N_PTS
NW = 32
PTS_W = BN // NW
RCH = 80
TN = 64


def _make_sc_gather_rows(Cp):
    mesh = plsc.VectorSubcoreMesh(core_axis_name="c", subcore_axis_name="s")
    ew = PTS_W * K

    @functools.partial(
        pl.kernel, mesh=mesh,
        compiler_params=pltpu.CompilerParams(
            needs_layout_passes=False, use_tc_tiling_on_sc=False),
        out_type=jax.ShapeDtypeStruct((BN * K, Cp), jnp.float32),
        scratch_types=[
            pltpu.VMEM((ew,), jnp.int32),
            pltpu.VMEM((RCH, Cp), jnp.float32),
            pltpu.VMEM((RCH, Cp), jnp.float32),
            pltpu.SemaphoreType.DMA,
            pltpu.SemaphoreType.DMA,
        ],
    )
    def sc_gather(idx_hbm, tab_hbm, g_hbm, idx_v, rows0_v, rows1_v, sem0, sem1):
        wid = lax.axis_index("s") * 2 + lax.axis_index("c")
        ebase = wid * ew
        pltpu.sync_copy(idx_hbm.at[pl.ds(ebase, ew)], idx_v)
        nch = ew // RCH
        rows = (rows0_v, rows1_v)
        sems = (sem0, sem1)

        def gath(c, buf):
            return pltpu.async_copy(
                tab_hbm.at[idx_v.at[pl.ds(c * RCH, RCH)]], rows[buf], sems[buf])

        def drain(buf):
            pltpu.make_async_copy(
                tab_hbm.at[idx_v.at[pl.ds(0, RCH)]], rows[buf], sems[buf]).wait()

        gath(0, 0)

        def _chunk(c, _):
            def body2(par):
                cc = c * 2 + par
                drain(par)
                nxt = cc + 1

                @pl.when(nxt < nch)
                def _():
                    gath(nxt, 1 - par)
                pltpu.sync_copy(rows[par], g_hbm.at[pl.ds(ebase + cc * RCH, RCH)])
            body2(0)
            body2(1)
            return _
        lax.fori_loop(0, nch // 2, _chunk, 0)

    return sc_gather


_SC_CACHE = {}


def _sc_gather_rows(idx_glob, table, Cp):
    if Cp not in _SC_CACHE:
        _SC_CACHE[Cp] = _make_sc_gather_rows(Cp)
    return _SC_CACHE[Cp](idx_glob, table)


def _make_edge_conv(Cp, O):
    grid = BN // TN

    def body(xi_ref, g_ref, w_ref, m_ref, s1_ref, s2_ref, c1_ref, c2_ref):
        step = pl.program_id(0)
        xi = xi_ref[...]
        gj = g_ref[...]
        diff = gj.reshape(TN, K, Cp) - xi[:, None, :]
        fa = diff.astype(jnp.bfloat16).reshape(TN * K, Cp)
        xb = jnp.broadcast_to(
            xi.astype(jnp.bfloat16)[:, None, :], (TN, K, Cp)).reshape(TN * K, Cp)
        f = jnp.concatenate([fa, xb], axis=1)
        w = w_ref[...]
        y = jax.lax.dot_general(
            f, w, (((1,), (0,)), ((), ())),
            preferred_element_type=jnp.float32)
        m_ref[...] = jnp.max(y.reshape(TN, K, O), axis=1)
        ps = jnp.sum(y, axis=0)[None]
        psq = jnp.sum(y * y, axis=0)[None]

        @pl.when(step == 0)
        def _():
            s1_ref[...] = ps
            s2_ref[...] = psq
            c1_ref[...] = jnp.zeros_like(ps)
            c2_ref[...] = jnp.zeros_like(ps)

        @pl.when(step != 0)
        def _():
            def acc(s_ref, c_ref, v):
                v = v + c_ref[...]
                s = s_ref[...]
                t = s + v
                c_ref[...] = (s - t) + v
                s_ref[...] = t
            acc(s1_ref, c1_ref, ps)
            acc(s2_ref, c2_ref, psq)

    return pl.pallas_call(
        body,
        grid=(grid,),
        in_specs=[
            pl.BlockSpec((TN, Cp), lambda i: (i, 0)),
            pl.BlockSpec((TN * K, Cp), lambda i: (i, 0)),
            pl.BlockSpec((2 * Cp, O), lambda i: (0, 0)),
        ],
        out_specs=[
            pl.BlockSpec((TN, O), lambda i: (i, 0)),
            pl.BlockSpec((1, O), lambda i: (0, 0)),
            pl.BlockSpec((1, O), lambda i: (0, 0)),
        ],
        out_shape=[
            jax.ShapeDtypeStruct((BN, O), jnp.float32),
            jax.ShapeDtypeStruct((1, O), jnp.float32),
            jax.ShapeDtypeStruct((1, O), jnp.float32),
        ],
        scratch_shapes=[
            pltpu.VMEM((1, O), jnp.float32),
            pltpu.VMEM((1, O), jnp.float32),
        ],
    )


def _make_finalize(O):
    grid = BN // 512

    def body(m_ref, mu_ref, var_ref, g_ref, b_ref, o_ref, t_ref):
        mean = mu_ref[...]
        scale = g_ref[...] / jnp.sqrt(var_ref[...] + EPS)
        shift = b_ref[...] - mean * scale
        v = m_ref[...] * scale + shift
        v = jnp.where(v > 0, v, 0.2 * v)
        o_ref[...] = v
        t_ref[...] = jnp.transpose(v)[None]

    return pl.pallas_call(
        body,
        grid=(grid,),
        in_specs=[
            pl.BlockSpec((512, O), lambda i: (i, 0)),
            pl.BlockSpec((1, O), lambda i: (0, 0)),
            pl.BlockSpec((1, O), lambda i: (0, 0)),
            pl.BlockSpec((1, O), lambda i: (0, 0)),
            pl.BlockSpec((1, O), lambda i: (0, 0)),
        ],
        out_specs=[
            pl.BlockSpec((512, O), lambda i: (i, 0)),
            pl.BlockSpec((1, O, 512), lambda i: (i // (N_PTS // 512), 0,
                                                 i % (N_PTS // 512))),
        ],
        out_shape=[
            jax.ShapeDtypeStruct((BN, O), jnp.float32),
            jax.ShapeDtypeStruct((NB, O, N_PTS), jnp.float32),
        ],
    )


_EC_CACHE = {}


def _edge_conv(xt_pad, G, w_t, g, b, Cp, O):
    if (Cp, O) not in _EC_CACHE:
        _EC_CACHE[(Cp, O)] = (_make_edge_conv(Cp, O), _make_finalize(O))
    conv, fin = _EC_CACHE[(Cp, O)]
    M, s1, s2 = conv(xt_pad, G, w_t)
    E = float(BN * K)
    mean = s1 / E
    var = s2 / E - mean * mean
    return fin(M, mean, var, g.reshape(1, O), b.reshape(1, O))


def _knn_idx(x):
    inner = -2.0 * jnp.einsum('bcn,bcm->bnm', x, x)
    xx = jnp.sum(x * x, axis=1)
    pairwise = -xx[:, :, None] - inner - xx[:, None, :]
    seg = pairwise.reshape(NB, N_PTS, N_PTS // 16, 16)
    segmax = seg.max(axis=-1)
    _, sidx = jax.lax.top_k(segmax, 24)
    cand = jnp.take_along_axis(seg, sidx[..., None], axis=2)
    _, ci = jax.lax.top_k(cand.reshape(NB, N_PTS, 24 * 16), K)
    seg_of = jnp.take_along_axis(sidx, ci // 16, axis=2)
    return seg_of * 16 + (ci % 16)


def _edge_block(xt_pad, x_cm, W, g, b, C, Cp):
    O = W.shape[0]
    idx = _knn_idx(x_cm)
    boffs = (jnp.arange(NB, dtype=jnp.int32) * N_PTS)[:, None, None]
    idx_glob = (idx + boffs).reshape(-1)
    G = _sc_gather_rows(idx_glob, xt_pad.reshape(BN, Cp), Cp)
    Wd, Wc = W[:, :C], W[:, C:]
    pad = Cp - C
    if pad:
        Wd = jnp.pad(Wd, ((0, 0), (0, pad)))
        Wc = jnp.pad(Wc, ((0, 0), (0, pad)))
    w_t = jnp.concatenate([Wd.T, Wc.T], axis=0).astype(jnp.bfloat16)
    return _edge_conv(xt_pad.reshape(BN, Cp), G, w_t, g, b, Cp, O)


def _lrelu(x):
    return jnp.where(x > 0, x, 0.2 * x)


def _edge_block_exact(xt_pad, x_cm, W, g, b, C, Cp):
    O = W.shape[0]
    idx = _knn_idx(x_cm)
    boffs = (jnp.arange(NB, dtype=jnp.int32) * N_PTS)[:, None, None]
    idx_glob = (idx + boffs).reshape(-1)
    G = _sc_gather_rows(idx_glob, xt_pad.reshape(BN, Cp), Cp)
    feature = G.reshape(NB, N_PTS, K, Cp)[:, :, :, :C]
    center = jnp.broadcast_to(
        xt_pad[:, :, None, :C], (NB, N_PTS, K, C))
    f = jnp.concatenate([feature - center, center], axis=-1)
    f = jnp.transpose(f, (0, 3, 1, 2))
    y = jnp.einsum('oc,bcnk->bonk', W, f)
    mean = jnp.mean(y, axis=(0, 2, 3), keepdims=True)
    var = jnp.var(y, axis=(0, 2, 3), keepdims=True)
    xhat = (y - mean) / jnp.sqrt(var + EPS)
    yn = xhat * g.reshape(1, O, 1, 1) + b.reshape(1, O, 1, 1)
    out_cm = jnp.max(_lrelu(yn), axis=-1)
    return jnp.transpose(out_cm, (0, 2, 1)), out_cm


def _bn_pw(x, g, b):
    mean = jnp.mean(x, axis=(0, 1), keepdims=True)
    var = jnp.var(x, axis=(0, 1), keepdims=True)
    return (x - mean) / jnp.sqrt(var + EPS) * g[None, None, :] + b[None, None, :]


def kernel(x, W1, g1, b1, W2, g2, b2, W3, g3, b3, W4, g4, b4, gcat, bcat,
           Wg, gg, bg, Wc1, gc1, bc1, Wc2, gc2, bc2, Wc3, bc3):
    B, C, N = x.shape
    xt = jnp.transpose(x, (0, 2, 1))
    xt_pad = jnp.pad(xt, ((0, 0), (0, 0), (0, 16 - C)))
    x1, x1_cm = _edge_block_exact(xt_pad, x, W1, g1, b1, C, 16)
    x2, x2_cm = _edge_block_exact(x1, x1_cm, W2, g2, b2, 64, 64)
    x3, x3_cm = _edge_block_exact(x2, x2_cm, W3, g3, b3, 64, 64)
    x4, _ = _edge_block(x3, x3_cm, W4, g4, b4, 128, 128)
    x4 = x4.reshape(B, N, 256)
    x_local = jnp.concatenate([x1, x2, x3, x4], axis=2)
    x_local = _bn_pw(x_local, gcat, bcat)
    xg = _lrelu(_bn_pw(jnp.einsum('bnc,oc->bno', x_local, Wg), gg, bg))
    x_global = jnp.max(xg, axis=1)
    x_cat = jnp.concatenate(
        [x_local, jnp.broadcast_to(x_global[:, None, :], (B, N, 1024))], axis=2)
    h = _lrelu(_bn_pw(jnp.einsum('bnc,oc->bno', x_cat, Wc1), gc1, bc1))
    h = _lrelu(_bn_pw(jnp.einsum('bnc,oc->bno', h, Wc2), gc2, bc2))
    logits = jnp.einsum('bnc,oc->bno', h, Wc3) + bc3[None, None, :]
    return (jnp.transpose(logits, (0, 2, 1)),
            jnp.transpose(x_local, (0, 2, 1)),
            x_global[:, :, None])


# speedup vs baseline: 5.9909x; 1.2505x over previous
import functools

import jax
import jax.numpy as jnp
from jax import lax
from jax.experimental import pallas as pl
from jax.experimental.pallas import tpu as pltpu
from jax.experimental.pallas import tpu_sc as plsc

K = 20
EPS = 1e-5
N_PTS = 2048
NB = 4
BN = NB * N_PTS
NW = 32
PTS_W = BN // NW
RCH = 80
TN = 64


def _make_sc_gather_rows(Cp):
    mesh = plsc.VectorSubcoreMesh(core_axis_name="c", subcore_axis_name="s")
    ew = PTS_W * K

    @functools.partial(
        pl.kernel, mesh=mesh,
        compiler_params=pltpu.CompilerParams(
            needs_layout_passes=False, use_tc_tiling_on_sc=False),
        out_type=jax.ShapeDtypeStruct((BN * K, Cp), jnp.float32),
        scratch_types=[
            pltpu.VMEM((ew,), jnp.int32),
            pltpu.VMEM((RCH, Cp), jnp.float32),
            pltpu.VMEM((RCH, Cp), jnp.float32),
            pltpu.SemaphoreType.DMA,
            pltpu.SemaphoreType.DMA,
        ],
    )
    def sc_gather(idx_hbm, tab_hbm, g_hbm, idx_v, rows0_v, rows1_v, sem0, sem1):
        wid = lax.axis_index("s") * 2 + lax.axis_index("c")
        ebase = wid * ew
        pltpu.sync_copy(idx_hbm.at[pl.ds(ebase, ew)], idx_v)
        nch = ew // RCH
        rows = (rows0_v, rows1_v)
        sems = (sem0, sem1)

        def gath(c, buf):
            return pltpu.async_copy(
                tab_hbm.at[idx_v.at[pl.ds(c * RCH, RCH)]], rows[buf], sems[buf])

        def drain(buf):
            pltpu.make_async_copy(
                tab_hbm.at[idx_v.at[pl.ds(0, RCH)]], rows[buf], sems[buf]).wait()

        gath(0, 0)

        def _chunk(c, _):
            def body2(par):
                cc = c * 2 + par
                drain(par)
                nxt = cc + 1

                @pl.when(nxt < nch)
                def _():
                    gath(nxt, 1 - par)
                pltpu.sync_copy(rows[par], g_hbm.at[pl.ds(ebase + cc * RCH, RCH)])
            body2(0)
            body2(1)
            return _
        lax.fori_loop(0, nch // 2, _chunk, 0)

    return sc_gather


_SC_CACHE = {}


def _sc_gather_rows(idx_glob, table, Cp):
    if Cp not in _SC_CACHE:
        _SC_CACHE[Cp] = _make_sc_gather_rows(Cp)
    return _SC_CACHE[Cp](idx_glob, table)


def _make_edge_conv(Cp, O):
    grid = BN // TN

    def body(xi_ref, g_ref, w_ref, m_ref, s1_ref, s2_ref, c1_ref, c2_ref):
        step = pl.program_id(0)
        xi = xi_ref[...]
        gj = g_ref[...]
        diff = gj.reshape(TN, K, Cp) - xi[:, None, :]
        fa = diff.astype(jnp.bfloat16).reshape(TN * K, Cp)
        xb = jnp.broadcast_to(
            xi.astype(jnp.bfloat16)[:, None, :], (TN, K, Cp)).reshape(TN * K, Cp)
        f = jnp.concatenate([fa, xb], axis=1)
        w = w_ref[...]
        y = jax.lax.dot_general(
            f, w, (((1,), (0,)), ((), ())),
            preferred_element_type=jnp.float32)
        m_ref[...] = jnp.max(y.reshape(TN, K, O), axis=1)
        ps = jnp.sum(y, axis=0)[None]
        psq = jnp.sum(y * y, axis=0)[None]

        @pl.when(step == 0)
        def _():
            s1_ref[...] = ps
            s2_ref[...] = psq
            c1_ref[...] = jnp.zeros_like(ps)
            c2_ref[...] = jnp.zeros_like(ps)

        @pl.when(step != 0)
        def _():
            def acc(s_ref, c_ref, v):
                v = v + c_ref[...]
                s = s_ref[...]
                t = s + v
                c_ref[...] = (s - t) + v
                s_ref[...] = t
            acc(s1_ref, c1_ref, ps)
            acc(s2_ref, c2_ref, psq)

    return pl.pallas_call(
        body,
        grid=(grid,),
        in_specs=[
            pl.BlockSpec((TN, Cp), lambda i: (i, 0)),
            pl.BlockSpec((TN * K, Cp), lambda i: (i, 0)),
            pl.BlockSpec((2 * Cp, O), lambda i: (0, 0)),
        ],
        out_specs=[
            pl.BlockSpec((TN, O), lambda i: (i, 0)),
            pl.BlockSpec((1, O), lambda i: (0, 0)),
            pl.BlockSpec((1, O), lambda i: (0, 0)),
        ],
        out_shape=[
            jax.ShapeDtypeStruct((BN, O), jnp.float32),
            jax.ShapeDtypeStruct((1, O), jnp.float32),
            jax.ShapeDtypeStruct((1, O), jnp.float32),
        ],
        scratch_shapes=[
            pltpu.VMEM((1, O), jnp.float32),
            pltpu.VMEM((1, O), jnp.float32),
        ],
    )


def _make_finalize(O):
    grid = BN // 512

    def body(m_ref, mu_ref, var_ref, g_ref, b_ref, o_ref, t_ref):
        mean = mu_ref[...]
        scale = g_ref[...] / jnp.sqrt(var_ref[...] + EPS)
        shift = b_ref[...] - mean * scale
        v = m_ref[...] * scale + shift
        v = jnp.where(v > 0, v, 0.2 * v)
        o_ref[...] = v
        t_ref[...] = jnp.transpose(v)[None]

    return pl.pallas_call(
        body,
        grid=(grid,),
        in_specs=[
            pl.BlockSpec((512, O), lambda i: (i, 0)),
            pl.BlockSpec((1, O), lambda i: (0, 0)),
            pl.BlockSpec((1, O), lambda i: (0, 0)),
            pl.BlockSpec((1, O), lambda i: (0, 0)),
            pl.BlockSpec((1, O), lambda i: (0, 0)),
        ],
        out_specs=[
            pl.BlockSpec((512, O), lambda i: (i, 0)),
            pl.BlockSpec((1, O, 512), lambda i: (i // (N_PTS // 512), 0,
                                                 i % (N_PTS // 512))),
        ],
        out_shape=[
            jax.ShapeDtypeStruct((BN, O), jnp.float32),
            jax.ShapeDtypeStruct((NB, O, N_PTS), jnp.float32),
        ],
    )


_EC_CACHE = {}


def _edge_conv(xt_pad, G, w_t, g, b, Cp, O):
    if (Cp, O) not in _EC_CACHE:
        _EC_CACHE[(Cp, O)] = (_make_edge_conv(Cp, O), _make_finalize(O))
    conv, fin = _EC_CACHE[(Cp, O)]
    M, s1, s2 = conv(xt_pad, G, w_t)
    E = float(BN * K)
    mean = s1 / E
    var = s2 / E - mean * mean
    return fin(M, mean, var, g.reshape(1, O), b.reshape(1, O))


def _knn_idx(x):
    inner = -2.0 * jnp.einsum('bcn,bcm->bnm', x, x)
    xx = jnp.sum(x * x, axis=1)
    pairwise = -xx[:, :, None] - inner - xx[:, None, :]
    SW, KEEP = 8, 24
    seg = pairwise.reshape(NB, N_PTS, N_PTS // SW, SW)
    segmax = seg.max(axis=-1)
    _, sidx = jax.lax.top_k(segmax, KEEP)
    cand = jnp.take_along_axis(seg, sidx[..., None], axis=2)
    _, ci = jax.lax.top_k(cand.reshape(NB, N_PTS, KEEP * SW), K)
    seg_of = jnp.take_along_axis(sidx, ci // SW, axis=2)
    return seg_of * SW + (ci % SW)


def _edge_block(xt_pad, x_cm, W, g, b, C, Cp):
    O = W.shape[0]
    idx = _knn_idx(x_cm)
    boffs = (jnp.arange(NB, dtype=jnp.int32) * N_PTS)[:, None, None]
    idx_glob = (idx + boffs).reshape(-1)
    G = _sc_gather_rows(idx_glob, xt_pad.reshape(BN, Cp), Cp)
    Wd, Wc = W[:, :C], W[:, C:]
    pad = Cp - C
    if pad:
        Wd = jnp.pad(Wd, ((0, 0), (0, pad)))
        Wc = jnp.pad(Wc, ((0, 0), (0, pad)))
    w_t = jnp.concatenate([Wd.T, Wc.T], axis=0).astype(jnp.bfloat16)
    return _edge_conv(xt_pad.reshape(BN, Cp), G, w_t, g, b, Cp, O)


def _lrelu(x):
    return jnp.where(x > 0, x, 0.2 * x)


def _edge_block_exact(xt_pad, x_cm, W, g, b, C, Cp):
    O = W.shape[0]
    idx = _knn_idx(x_cm)
    boffs = (jnp.arange(NB, dtype=jnp.int32) * N_PTS)[:, None, None]
    idx_glob = (idx + boffs).reshape(-1)
    G = _sc_gather_rows(idx_glob, xt_pad.reshape(BN, Cp), Cp)
    feature = G.reshape(NB, N_PTS, K, Cp)[:, :, :, :C]
    center = jnp.broadcast_to(
        xt_pad[:, :, None, :C], (NB, N_PTS, K, C))
    f = jnp.concatenate([feature - center, center], axis=-1)
    f = jnp.transpose(f, (0, 3, 1, 2))
    y = jnp.einsum('oc,bcnk->bonk', W, f)
    mean = jnp.mean(y, axis=(0, 2, 3), keepdims=True)
    var = jnp.var(y, axis=(0, 2, 3), keepdims=True)
    xhat = (y - mean) / jnp.sqrt(var + EPS)
    yn = xhat * g.reshape(1, O, 1, 1) + b.reshape(1, O, 1, 1)
    out_cm = jnp.max(_lrelu(yn), axis=-1)
    return jnp.transpose(out_cm, (0, 2, 1)), out_cm


def _bn_pw(x, g, b):
    mean = jnp.mean(x, axis=(0, 1), keepdims=True)
    var = jnp.var(x, axis=(0, 1), keepdims=True)
    return (x - mean) / jnp.sqrt(var + EPS) * g[None, None, :] + b[None, None, :]


def kernel(x, W1, g1, b1, W2, g2, b2, W3, g3, b3, W4, g4, b4, gcat, bcat,
           Wg, gg, bg, Wc1, gc1, bc1, Wc2, gc2, bc2, Wc3, bc3):
    B, C, N = x.shape
    xt = jnp.transpose(x, (0, 2, 1))
    xt_pad = jnp.pad(xt, ((0, 0), (0, 0), (0, 16 - C)))
    x1, x1_cm = _edge_block_exact(xt_pad, x, W1, g1, b1, C, 16)
    x2, x2_cm = _edge_block_exact(x1, x1_cm, W2, g2, b2, 64, 64)
    x3, x3_cm = _edge_block_exact(x2, x2_cm, W3, g3, b3, 64, 64)
    x4, _ = _edge_block(x3, x3_cm, W4, g4, b4, 128, 128)
    x4 = x4.reshape(B, N, 256)
    x_local = jnp.concatenate([x1, x2, x3, x4], axis=2)
    x_local = _bn_pw(x_local, gcat, bcat)
    xg = _lrelu(_bn_pw(jnp.einsum('bnc,oc->bno', x_local, Wg), gg, bg))
    x_global = jnp.max(xg, axis=1)
    x_cat = jnp.concatenate(
        [x_local, jnp.broadcast_to(x_global[:, None, :], (B, N, 1024))], axis=2)
    h = _lrelu(_bn_pw(jnp.einsum('bnc,oc->bno', x_cat, Wc1), gc1, bc1))
    h = _lrelu(_bn_pw(jnp.einsum('bnc,oc->bno', h, Wc2), gc2, bc2))
    logits = jnp.einsum('bnc,oc->bno', h, Wc3) + bc3[None, None, :]
    return (jnp.transpose(logits, (0, 2, 1)),
            jnp.transpose(x_local, (0, 2, 1)),
            x_global[:, :, None])
